# Initial kernel scaffold; baseline (speedup 1.0000x reference)
import jax
import jax.numpy as jnp
from jax.experimental import pallas as pl


def kernel(x, edge_index, ids, W_enc, b_enc, W_msg, b_msg, W_proc, b_proc, W_dec, b_dec):
    raise NotImplementedError("write your pallas kernel here")


# speedup vs baseline: 1.0882x; 1.0882x over previous
import functools

import jax
import jax.numpy as jnp
from jax import lax
from jax.experimental import pallas as pl
from jax.experimental.pallas import tpu as pltpu

N = 10000
E = 320000
G = 64
NW = 32
ROWS_W = 313
N_PAD = NW * ROWS_W


def _enc_body(x_ref, we_ref, be_ref, wm_ref, bm_ref, h_ref, hm_ref):
    h = jnp.maximum(
        jnp.dot(x_ref[...], we_ref[...], preferred_element_type=jnp.float32)
        + be_ref[...], 0.0)
    h_ref[...] = h
    hm_ref[...] = (
        jnp.dot(h, wm_ref[...], preferred_element_type=jnp.float32)
        + bm_ref[...])


def _encoder(x, W_enc, b_enc, W_msg, b_msg):
    blk = 2000
    grid = (N // blk,)
    return pl.pallas_call(
        _enc_body,
        grid=grid,
        in_specs=[
            pl.BlockSpec((blk, 128), lambda i: (i, 0)),
            pl.BlockSpec((128, 128), lambda i: (0, 0)),
            pl.BlockSpec((1, 128), lambda i: (0, 0)),
            pl.BlockSpec((128, 128), lambda i: (0, 0)),
            pl.BlockSpec((1, 128), lambda i: (0, 0)),
        ],
        out_specs=[
            pl.BlockSpec((blk, 128), lambda i: (i, 0)),
            pl.BlockSpec((blk, 128), lambda i: (i, 0)),
        ],
        out_shape=[
            jax.ShapeDtypeStruct((N, 128), jnp.float32),
            jax.ShapeDtypeStruct((N, 128), jnp.float32),
        ],
    )(x, W_enc, b_enc, W_msg, b_msg)


def _proc_body(a_ref, wp_ref, bp_ref, p_ref):
    p_ref[...] = jnp.maximum(
        jnp.dot(a_ref[...], wp_ref[...], preferred_element_type=jnp.float32)
        + bp_ref[...], 0.0)


def _processor(agg, W_proc, b_proc):
    rows = agg.shape[0]
    blk = 2504
    grid = (rows // blk,)
    return pl.pallas_call(
        _proc_body,
        grid=grid,
        in_specs=[
            pl.BlockSpec((blk, 128), lambda i: (i, 0)),
            pl.BlockSpec((128, 128), lambda i: (0, 0)),
            pl.BlockSpec((1, 128), lambda i: (0, 0)),
        ],
        out_specs=pl.BlockSpec((blk, 128), lambda i: (i, 0)),
        out_shape=jax.ShapeDtypeStruct((rows, 128), jnp.float32),
    )(agg, W_proc, b_proc)


def kernel(x, edge_index, ids, W_enc, b_enc, W_msg, b_msg, W_proc, b_proc,
           W_dec, b_dec):
    h, hm = _encoder(x, W_enc, b_enc.reshape(1, -1), W_msg,
                     b_msg.reshape(1, -1))
    src = edge_index[0]
    dst = edge_index[1]
    agg = jnp.maximum(
        jax.ops.segment_max(hm[src], dst, num_segments=N), 0.0)
    agg = jnp.pad(agg, ((0, N_PAD - N), (0, 0)))
    p = _processor(agg, W_proc, b_proc.reshape(1, -1))[:N]
    h_g = jnp.maximum(jax.ops.segment_max(h, ids, num_segments=G), 0.0)
    p_g = jnp.maximum(jax.ops.segment_max(p, ids, num_segments=G), 0.0)
    dec_in = jnp.concatenate([h_g, p_g], axis=1)
    y = (dec_in @ W_dec + b_dec).squeeze(-1)
    return y


# speedup vs baseline: 1.2062x; 1.1084x over previous
import functools

import jax
import jax.numpy as jnp
from jax import lax
from jax.experimental import pallas as pl
from jax.experimental.pallas import tpu as pltpu
from jax.experimental.pallas import tpu_sc as plsc

N = 10000
E = 320000
G = 64
NW = 32
ROWS_W = 320
N_PAD = NW * ROWS_W


def _enc_body(x_ref, we_ref, be_ref, wm_ref, bm_ref, h_ref, hm_ref):
    h = jnp.maximum(
        jnp.dot(x_ref[...], we_ref[...], preferred_element_type=jnp.float32)
        + be_ref[...], 0.0)
    h_ref[...] = h
    hm_ref[...] = (
        jnp.dot(h, wm_ref[...], preferred_element_type=jnp.float32)
        + bm_ref[...])


def _encoder(x, W_enc, b_enc, W_msg, b_msg):
    blk = 2000
    grid = (N // blk,)
    return pl.pallas_call(
        _enc_body,
        grid=grid,
        in_specs=[
            pl.BlockSpec((blk, 128), lambda i: (i, 0)),
            pl.BlockSpec((128, 128), lambda i: (0, 0)),
            pl.BlockSpec((1, 128), lambda i: (0, 0)),
            pl.BlockSpec((128, 128), lambda i: (0, 0)),
            pl.BlockSpec((1, 128), lambda i: (0, 0)),
        ],
        out_specs=[
            pl.BlockSpec((blk, 128), lambda i: (i, 0)),
            pl.BlockSpec((blk, 128), lambda i: (i, 0)),
        ],
        out_shape=[
            jax.ShapeDtypeStruct((N, 128), jnp.float32),
            jax.ShapeDtypeStruct((N, 128), jnp.float32),
        ],
    )(x, W_enc, b_enc, W_msg, b_msg)


def _proc_body(a_ref, wp_ref, bp_ref, p_ref):
    p_ref[...] = jnp.maximum(
        jnp.dot(a_ref[...], wp_ref[...], preferred_element_type=jnp.float32)
        + bp_ref[...], 0.0)


def _processor(agg, W_proc, b_proc):
    rows = agg.shape[0]
    blk = 2560
    grid = (rows // blk,)
    return pl.pallas_call(
        _proc_body,
        grid=grid,
        in_specs=[
            pl.BlockSpec((blk, 128), lambda i: (i, 0)),
            pl.BlockSpec((128, 128), lambda i: (0, 0)),
            pl.BlockSpec((1, 128), lambda i: (0, 0)),
        ],
        out_specs=pl.BlockSpec((blk, 128), lambda i: (i, 0)),
        out_shape=jax.ShapeDtypeStruct((rows, 128), jnp.float32),
    )(agg, W_proc, b_proc)


_CH = 2000
_NCHUNK = E // _CH
_NVEC = _CH // 16
_BG = 64
_LD_DUMP = ROWS_W


def _edge_body(hm_hbm, src_hbm, dst_hbm, out_hbm,
               acc, rows, dstc, srcc, pkd, srcb, ldb, sem):
    cid = lax.axis_index("c")
    sid = lax.axis_index("s")
    wid = sid * 2 + cid
    base = wid * ROWS_W
    zero16 = jnp.zeros((16,), jnp.float32)
    iota = lax.iota(jnp.int32, 16)

    def _init(i, c):
        for j in range(8):
            acc[i, pl.ds(16 * j, 16)] = zero16
        return c
    lax.fori_loop(0, ROWS_W + 1, _init, 0, unroll=False)

    def _chunk(ci, _):
        pltpu.sync_copy(dst_hbm.at[pl.ds(ci * _CH, _CH)], dstc)
        pltpu.sync_copy(src_hbm.at[pl.ds(ci * _CH, _CH)], srcc)

        def _scan(v, cur):
            dv = dstc[pl.ds(v * 16, 16)]
            sv = srcc[pl.ds(v * 16, 16)]
            t = dv - base
            m = (t >= 0) & (t < ROWS_W)
            packed = (t << 14) | sv
            mi = m.astype(jnp.int32)
            incl = plsc.cumsum(mi)
            pos = cur + (incl - mi)
            plsc.store_scatter(pkd, [pos], packed, mask=m)
            cnt = jnp.max(incl)
            return cur + cnt
        cur = lax.fori_loop(0, _NVEC, _scan, 0, unroll=False)

        padv = (_LD_DUMP << 14) | iota
        for k in range(_BG // 16):
            pkd[pl.ds(cur + 16 * k, 16)] = padv

        nsb = (cur + _BG - 1) >> 6

        def _unpack(g, _):
            v = pkd[pl.ds(g * 16, 16)]
            srcb[pl.ds(g * 16, 16)] = v & 0x3FFF
            ldb[pl.ds(g * 16, 16)] = lax.shift_right_logical(v, 14)
            return 0
        lax.fori_loop(0, nsb * (_BG // 16), _unpack, 0, unroll=False)

        def _batch(b, _):
            pltpu.async_copy(
                hm_hbm.at[srcb.at[pl.ds(b * _BG, _BG)]], rows, sem).wait()
            for g in range(_BG // 16):
                ldv = ldb[pl.ds(b * _BG + g * 16, 16)]
                for k in range(16):
                    ld = jnp.max(jnp.where(iota == k, ldv, 0))
                    e = g * 16 + k
                    for j in range(8):
                        a = acc[ld, pl.ds(16 * j, 16)]
                        r = rows[e, pl.ds(16 * j, 16)]
                        acc[ld, pl.ds(16 * j, 16)] = jnp.maximum(a, r)
            return 0
        lax.fori_loop(0, nsb, _batch, 0, unroll=False)
        return 0

    lax.fori_loop(0, _NCHUNK, _chunk, 0, unroll=False)
    pltpu.sync_copy(acc.at[pl.ds(0, ROWS_W)],
                    out_hbm.at[pl.ds(base, ROWS_W)])


def _edge_segmax(hm, src, dst):
    mesh = plsc.VectorSubcoreMesh(core_axis_name="c", subcore_axis_name="s")
    f = pl.kernel(
        _edge_body,
        mesh=mesh,
        compiler_params=pltpu.CompilerParams(needs_layout_passes=False),
        out_type=jax.ShapeDtypeStruct((N_PAD, 128), jnp.float32),
        scratch_types=[
            pltpu.VMEM((ROWS_W + 1, 128), jnp.float32),
            pltpu.VMEM((_BG, 128), jnp.float32),
            pltpu.VMEM((_CH,), jnp.int32),
            pltpu.VMEM((_CH,), jnp.int32),
            pltpu.VMEM((_CH + _BG + 16,), jnp.int32),
            pltpu.VMEM((_CH + _BG + 16,), jnp.int32),
            pltpu.VMEM((_CH + _BG + 16,), jnp.int32),
            pltpu.SemaphoreType.DMA,
        ],
    )
    return f(hm, src, dst)


def kernel(x, edge_index, ids, W_enc, b_enc, W_msg, b_msg, W_proc, b_proc,
           W_dec, b_dec):
    h, hm = _encoder(x, W_enc, b_enc.reshape(1, -1), W_msg,
                     b_msg.reshape(1, -1))
    src = edge_index[0]
    dst = edge_index[1]
    agg = _edge_segmax(hm, src, dst)
    p = _processor(agg, W_proc, b_proc.reshape(1, -1))[:N]
    h_g = jnp.maximum(jax.ops.segment_max(h, ids, num_segments=G), 0.0)
    p_g = jnp.maximum(jax.ops.segment_max(p, ids, num_segments=G), 0.0)
    dec_in = jnp.concatenate([h_g, p_g], axis=1)
    y = (dec_in @ W_dec + b_dec).squeeze(-1)
    return y


# speedup vs baseline: 1.2502x; 1.0365x over previous
import functools

import jax
import jax.numpy as jnp
from jax import lax
from jax.experimental import pallas as pl
from jax.experimental.pallas import tpu as pltpu
from jax.experimental.pallas import tpu_sc as plsc

N = 10000
E = 320000
G = 64
NW = 32
ROWS_W = 320
N_PAD = NW * ROWS_W


def _enc_body(x_ref, we_ref, be_ref, wm_ref, bm_ref, h_ref, hm_ref):
    h = jnp.maximum(
        jnp.dot(x_ref[...], we_ref[...], preferred_element_type=jnp.float32)
        + be_ref[...], 0.0)
    h_ref[...] = h
    hm_ref[...] = (
        jnp.dot(h, wm_ref[...], preferred_element_type=jnp.float32)
        + bm_ref[...])


def _encoder(x, W_enc, b_enc, W_msg, b_msg):
    blk = 2000
    grid = (N // blk,)
    return pl.pallas_call(
        _enc_body,
        grid=grid,
        in_specs=[
            pl.BlockSpec((blk, 128), lambda i: (i, 0)),
            pl.BlockSpec((128, 128), lambda i: (0, 0)),
            pl.BlockSpec((1, 128), lambda i: (0, 0)),
            pl.BlockSpec((128, 128), lambda i: (0, 0)),
            pl.BlockSpec((1, 128), lambda i: (0, 0)),
        ],
        out_specs=[
            pl.BlockSpec((blk, 128), lambda i: (i, 0)),
            pl.BlockSpec((blk, 128), lambda i: (i, 0)),
        ],
        out_shape=[
            jax.ShapeDtypeStruct((N, 128), jnp.float32),
            jax.ShapeDtypeStruct((N, 128), jnp.float32),
        ],
    )(x, W_enc, b_enc, W_msg, b_msg)


def _proc_body(a_ref, wp_ref, bp_ref, p_ref):
    p_ref[...] = jnp.maximum(
        jnp.dot(a_ref[...], wp_ref[...], preferred_element_type=jnp.float32)
        + bp_ref[...], 0.0)


def _processor(agg, W_proc, b_proc):
    rows = agg.shape[0]
    blk = 2560
    grid = (rows // blk,)
    return pl.pallas_call(
        _proc_body,
        grid=grid,
        in_specs=[
            pl.BlockSpec((blk, 128), lambda i: (i, 0)),
            pl.BlockSpec((128, 128), lambda i: (0, 0)),
            pl.BlockSpec((1, 128), lambda i: (0, 0)),
        ],
        out_specs=pl.BlockSpec((blk, 128), lambda i: (i, 0)),
        out_shape=jax.ShapeDtypeStruct((rows, 128), jnp.float32),
    )(agg, W_proc, b_proc)


_CH = 2000
_NVEC = _CH // 16
_BG = 64
_MASK14 = (1 << 14) - 1


def _phase(*, nchunk, key_hbm, val_hbm, table_hbm, acc, nrows, base,
           rows, keyc, valc, pkd, srcb, ldb, sem, iota, use_pos):
    dump = nrows
    full15 = jnp.full((16,), 15, jnp.int32)

    def _chunk(ci, _):
        pltpu.sync_copy(key_hbm.at[pl.ds(ci * _CH, _CH)], keyc)
        if not use_pos:
            pltpu.sync_copy(val_hbm.at[pl.ds(ci * _CH, _CH)], valc)

        def _scan(v, curv):
            kv = keyc[pl.ds(v * 16, 16)]
            t = kv - base
            m = (t >= 0) & (t < nrows)
            if use_pos:
                pay = (ci * _CH + v * 16) + iota
            else:
                pay = valc[pl.ds(v * 16, 16)]
            packed = (t << 14) | pay
            mi = m.astype(jnp.int32)
            incl = plsc.cumsum(mi)
            pos = curv + (incl - mi)
            plsc.store_scatter(pkd, [pos], packed, mask=m)
            return curv + jnp.take(incl, full15)
        curv = lax.fori_loop(0, _NVEC, _scan, jnp.zeros((16,), jnp.int32),
                             unroll=8)

        padv = (dump << 14) | iota
        for k in range(_BG // 16):
            plsc.store_scatter(pkd, [curv + (16 * k) + iota], padv)

        nsb = (jnp.max(curv) + _BG - 1) >> 6

        def _unpack(g, _):
            v = pkd[pl.ds(g * 16, 16)]
            srcb[pl.ds(g * 16, 16)] = v & _MASK14
            ldb[pl.ds(g * 16, 16)] = lax.shift_right_logical(v, 14)
            return 0
        lax.fori_loop(0, nsb * (_BG // 16), _unpack, 0, unroll=False)

        def _batch(b, _):
            pltpu.async_copy(
                table_hbm.at[srcb.at[pl.ds(b * _BG, _BG)]], rows, sem).wait()
            for g in range(_BG // 16):
                ldv = ldb[pl.ds(b * _BG + g * 16, 16)]
                for k in range(16):
                    ld = jnp.max(jnp.where(iota == k, ldv, 0))
                    e = g * 16 + k
                    for j in range(8):
                        a = acc[ld, pl.ds(16 * j, 16)]
                        r = rows[e, pl.ds(16 * j, 16)]
                        acc[ld, pl.ds(16 * j, 16)] = jnp.maximum(a, r)
            return 0
        lax.fori_loop(0, nsb, _batch, 0, unroll=False)
        return 0

    lax.fori_loop(0, nchunk, _chunk, 0, unroll=False)


def _zero_rows(acc, n):
    zero16 = jnp.zeros((16,), jnp.float32)

    def _init(i, c):
        for j in range(8):
            acc[i, pl.ds(16 * j, 16)] = zero16
        return c
    lax.fori_loop(0, n, _init, 0, unroll=False)


def _main_body(hm_hbm, h_hbm, src_hbm, dst_hbm, ids_hbm, agg_out, hg_out,
               acc, rows, keyc, valc, pkd, srcb, ldb, sem):
    cid = lax.axis_index("c")
    sid = lax.axis_index("s")
    wid = sid * 2 + cid
    iota = lax.iota(jnp.int32, 16)
    kw = dict(rows=rows, keyc=keyc, valc=valc, pkd=pkd, srcb=srcb, ldb=ldb,
              sem=sem, iota=iota)

    _zero_rows(acc, ROWS_W + 1)
    _phase(nchunk=E // _CH, key_hbm=dst_hbm, val_hbm=src_hbm,
           table_hbm=hm_hbm, acc=acc, nrows=ROWS_W, base=wid * ROWS_W,
           use_pos=False, **kw)
    pltpu.sync_copy(acc.at[pl.ds(0, ROWS_W)],
                    agg_out.at[pl.ds(wid * ROWS_W, ROWS_W)])

    _zero_rows(acc, 3)
    _phase(nchunk=N // _CH, key_hbm=ids_hbm, val_hbm=None,
           table_hbm=h_hbm, acc=acc, nrows=2, base=wid * 2,
           use_pos=True, **kw)
    pltpu.sync_copy(acc.at[pl.ds(0, 8)], hg_out.at[pl.ds(wid * 8, 8)])


def _sc_main(hm, h, src, dst, ids):
    mesh = plsc.VectorSubcoreMesh(core_axis_name="c", subcore_axis_name="s")
    f = pl.kernel(
        _main_body,
        mesh=mesh,
        compiler_params=pltpu.CompilerParams(needs_layout_passes=False),
        out_type=[
            jax.ShapeDtypeStruct((N_PAD, 128), jnp.float32),
            jax.ShapeDtypeStruct((NW * 8, 128), jnp.float32),
        ],
        scratch_types=[
            pltpu.VMEM((ROWS_W + 1, 128), jnp.float32),
            pltpu.VMEM((_BG, 128), jnp.float32),
            pltpu.VMEM((_CH,), jnp.int32),
            pltpu.VMEM((_CH,), jnp.int32),
            pltpu.VMEM((_CH + _BG + 16,), jnp.int32),
            pltpu.VMEM((_CH + _BG + 16,), jnp.int32),
            pltpu.VMEM((_CH + _BG + 16,), jnp.int32),
            pltpu.SemaphoreType.DMA,
        ],
    )
    return f(hm, h, src, dst, ids)


def _pool_decode_body(p_hbm, ids_hbm, hg_hbm, wd_hbm, y_out,
                      acc, rows, keyc, valc, pkd, srcb, ldb, hgv, wdv, ys,
                      sem):
    cid = lax.axis_index("c")
    sid = lax.axis_index("s")
    wid = sid * 2 + cid
    iota = lax.iota(jnp.int32, 16)

    _zero_rows(acc, 3)
    _phase(nchunk=N // _CH, key_hbm=ids_hbm, val_hbm=None,
           table_hbm=p_hbm, acc=acc, nrows=2, base=wid * 2,
           use_pos=True, rows=rows, keyc=keyc, valc=valc, pkd=pkd,
           srcb=srcb, ldb=ldb, sem=sem, iota=iota)

    pltpu.sync_copy(hg_hbm.at[pl.ds(wid * 8, 8)], hgv)
    pltpu.sync_copy(wd_hbm, wdv)
    for g in range(2):
        s = jnp.zeros((16,), jnp.float32)
        for j in range(8):
            s = (s + hgv[g, pl.ds(16 * j, 16)] * wdv[0, pl.ds(16 * j, 16)]
                 + acc[g, pl.ds(16 * j, 16)] * wdv[1, pl.ds(16 * j, 16)])
        y = jnp.sum(s)
        ys[g, pl.ds(0, 16)] = jnp.where(iota == 0, y,
                                        jnp.zeros((16,), jnp.float32))
    for g in range(2, 8):
        ys[g, pl.ds(0, 16)] = jnp.zeros((16,), jnp.float32)
    pltpu.sync_copy(ys, y_out.at[pl.ds(wid * 8, 8)])


def _sc_pool_decode(p, ids, hg, wd2):
    mesh = plsc.VectorSubcoreMesh(core_axis_name="c", subcore_axis_name="s")
    f = pl.kernel(
        _pool_decode_body,
        mesh=mesh,
        compiler_params=pltpu.CompilerParams(needs_layout_passes=False),
        out_type=jax.ShapeDtypeStruct((NW * 8, 16), jnp.float32),
        scratch_types=[
            pltpu.VMEM((3, 128), jnp.float32),
            pltpu.VMEM((_BG, 128), jnp.float32),
            pltpu.VMEM((_CH,), jnp.int32),
            pltpu.VMEM((_CH,), jnp.int32),
            pltpu.VMEM((_CH + _BG + 16,), jnp.int32),
            pltpu.VMEM((_CH + _BG + 16,), jnp.int32),
            pltpu.VMEM((_CH + _BG + 16,), jnp.int32),
            pltpu.VMEM((8, 128), jnp.float32),
            pltpu.VMEM((2, 128), jnp.float32),
            pltpu.VMEM((8, 16), jnp.float32),
            pltpu.SemaphoreType.DMA,
        ],
    )
    return f(p, ids, hg, wd2)


def kernel(x, edge_index, ids, W_enc, b_enc, W_msg, b_msg, W_proc, b_proc,
           W_dec, b_dec):
    h, hm = _encoder(x, W_enc, b_enc.reshape(1, -1), W_msg,
                     b_msg.reshape(1, -1))
    src = edge_index[0]
    dst = edge_index[1]
    agg, hg = _sc_main(hm, h, src, dst, ids)
    p = _processor(agg, W_proc, b_proc.reshape(1, -1))
    ystage = _sc_pool_decode(p, ids, hg, W_dec.reshape(2, 128))
    y = ystage.reshape(NW, 8, 16)[:, :2, 0].reshape(G) + b_dec
    return y
